# baseline (speedup 1.0000x reference)
import jax
import jax.numpy as jnp
from jax import lax
from jax.experimental import pallas as pl
from jax.experimental.pallas import tpu as pltpu
from jax.experimental.pallas import tpu_sc as plsc

V_DIM = 1000000
EMB = 32
BATCH = 16384
CTX = 20
ROWS_PER_LINE = 128 // EMB
NC, NS = 2, 16
NW = NC * NS
S_PER_W = BATCH // NW
CHUNK = 32
N_CHUNKS = S_PER_W // CHUNK
ROWS = CHUNK * CTX
OUT_LINES = CHUNK // ROWS_PER_LINE
INV_CTX = float(1.0 / CTX)


def _sc_body(idx_hbm, off_hbm, tab_hbm, out_hbm, idx_v, off_v, rows_v,
             out_v, sem):
    wid = lax.axis_index("s") * NC + lax.axis_index("c")
    base_out = wid * S_PER_W

    def chunk_body(ci, carry):
        off_out = base_out + ci * CHUNK
        off_idx = off_out * CTX
        pltpu.sync_copy(idx_hbm.at[pl.ds(off_idx, ROWS)], idx_v)
        pltpu.sync_copy(off_hbm.at[pl.ds(off_idx, ROWS)], off_v)
        pltpu.async_copy(tab_hbm.at[idx_v], rows_v, sem).wait()

        def out_body(o, c2):
            base = o * CTX
            va = off_v[pl.ds(base, 16)]
            vb = off_v[pl.ds(base + CTX - 16, 16)]
            starts = ([va[c] for c in range(16)]
                      + [vb[32 - CTX + c] for c in range(CTX - 16)])
            oline = o // ROWS_PER_LINE
            ocol = (o % ROWS_PER_LINE) * EMB
            for h in range(EMB // 16):
                vals = [
                    plsc.bitcast(
                        rows_v[base + c, pl.ds(starts[c] + h * 16, 16)],
                        jnp.float32)
                    for c in range(CTX)
                ]
                while len(vals) > 1:
                    vals = [a + b for a, b in zip(vals[::2], vals[1::2])] + (
                        [vals[-1]] if len(vals) % 2 else [])
                out_v[oline, pl.ds(ocol + h * 16, 16)] = plsc.bitcast(
                    vals[0] * INV_CTX, jnp.int32)
            return c2

        lax.fori_loop(0, CHUNK, out_body, 0)
        pltpu.sync_copy(
            out_v,
            out_hbm.at[pl.ds(off_out // ROWS_PER_LINE, OUT_LINES)])
        return carry

    lax.fori_loop(0, N_CHUNKS, chunk_body, 0)


@jax.jit
def _cbow(x_flat, tab):
    tab_lines = lax.bitcast_convert_type(tab, jnp.int32).reshape(
        V_DIM // ROWS_PER_LINE, 128)
    idx = lax.shift_right_logical(x_flat, 2)
    off = lax.mul(lax.bitwise_and(x_flat, 3), EMB)
    mesh = plsc.VectorSubcoreMesh(core_axis_name="c", subcore_axis_name="s")
    f = pl.kernel(
        _sc_body,
        out_type=jax.ShapeDtypeStruct((BATCH // ROWS_PER_LINE, 128),
                                      jnp.int32),
        mesh=mesh,
        scratch_types=[
            pltpu.VMEM((ROWS,), jnp.int32),
            pltpu.VMEM((ROWS,), jnp.int32),
            pltpu.VMEM((ROWS, 128), jnp.int32),
            pltpu.VMEM((OUT_LINES, 128), jnp.int32),
            pltpu.SemaphoreType.DMA,
        ],
        compiler_params=pltpu.CompilerParams(
            use_tc_tiling_on_sc=False, needs_layout_passes=False),
    )
    out_lines = f(idx, off, tab_lines)
    return lax.bitcast_convert_type(
        out_lines.reshape(BATCH, EMB), jnp.float32)


def kernel(x, emb_weight):
    return _cbow(x.reshape(-1), emb_weight)
